# baseline (speedup 1.0000x reference)
import jax
import jax.numpy as jnp
from jax import lax
from jax.experimental import pallas as pl
from jax.experimental.pallas import tpu as pltpu
from jax.experimental.pallas import tpu_sc as plsc

N_VOCAB_ = 49408
N_EMBD_ = 768
N_TOKEN_ = 77
BATCH_ = 1024

NC = 2
NS = 16
LANES = 16
NW = NC * NS

ROWS_PER_W = BATCH_ * N_TOKEN_ // NW
CH = 32
NCHUNK = ROWS_PER_W // CH
NPAIR = (NCHUNK + 1) // 2
D_SLICES = N_EMBD_ // LANES


def _body(tok_hbm, tab_hbm, pos_hbm, out_hbm, idx_v, pos_v, buf0, buf1, sem0, sem1):
    wid = lax.axis_index("s") * NC + lax.axis_index("c")
    row0w = wid * ROWS_PER_W

    pltpu.sync_copy(tok_hbm.at[wid], idx_v)
    pltpu.sync_copy(pos_hbm, pos_v)

    def start_gather(c, buf, sem):
        pltpu.async_copy(tab_hbm.at[idx_v.at[c]], buf, sem)

    def wait_gather(c, buf, sem):
        pltpu.make_async_copy(tab_hbm.at[idx_v.at[c]], buf, sem).wait()

    def process(c, buf):
        t0 = lax.rem(c * CH, N_TOKEN_)

        def row_body(r, _):
            t = lax.rem(t0 + r, N_TOKEN_)
            for j in range(D_SLICES):
                sl = pl.ds(j * LANES, LANES)
                plsc.addupdate(buf.at[r, sl], pos_v[t, sl])
            return 0

        lax.fori_loop(0, CH, row_body, 0)
        pltpu.sync_copy(buf, out_hbm.at[pl.ds(row0w + c * CH, CH)])

    start_gather(0, buf0, sem0)

    def pair_body(p, _):
        c0 = p * 2

        @pl.when(c0 + 1 < NCHUNK)
        def _():
            start_gather(c0 + 1, buf1, sem1)

        wait_gather(c0, buf0, sem0)
        process(c0, buf0)

        @pl.when(c0 + 2 < NCHUNK)
        def _():
            start_gather(c0 + 2, buf0, sem0)

        @pl.when(c0 + 1 < NCHUNK)
        def _():
            wait_gather(c0 + 1, buf1, sem1)
            process(c0 + 1, buf1)

        return 0

    lax.fori_loop(0, NPAIR, pair_body, 0)


@jax.jit
def kernel(tokens, embedding_token, embedding_posicao):
    mesh = plsc.VectorSubcoreMesh(core_axis_name="c", subcore_axis_name="s")
    tok3 = tokens.astype(jnp.int32).reshape(NW, NCHUNK, CH)
    out = pl.kernel(
        _body,
        out_type=jax.ShapeDtypeStruct((BATCH_ * N_TOKEN_, N_EMBD_), jnp.float32),
        mesh=mesh,
        scratch_types=[
            pltpu.VMEM((NCHUNK, CH), jnp.int32),
            pltpu.VMEM((N_TOKEN_, N_EMBD_), jnp.float32),
            pltpu.VMEM((CH, N_EMBD_), jnp.float32),
            pltpu.VMEM((CH, N_EMBD_), jnp.float32),
            pltpu.SemaphoreType.DMA,
            pltpu.SemaphoreType.DMA,
        ],
    )(tok3, embedding_token, embedding_posicao)
    return out.reshape(BATCH_, N_TOKEN_, N_EMBD_)


# speedup vs baseline: 1.3507x; 1.3507x over previous
import jax
import jax.numpy as jnp
from jax import lax
from jax.experimental import pallas as pl
from jax.experimental.pallas import tpu as pltpu
from jax.experimental.pallas import tpu_sc as plsc

N_VOCAB_ = 49408
N_EMBD_ = 768
N_TOKEN_ = 77
BATCH_ = 1024

NC = 2
NS = 16
LANES = 16
NW = NC * NS

B_PER_W = BATCH_ // NW
CP = 24
CT = 8
CW = 5
D_SLICES = N_EMBD_ // LANES
RG = 4


def _body(idxA_hbm, idxB_hbm, idxC_hbm, idxT_hbm, tab_hbm, pos_hbm, out_hbm,
          idxA_v, idxB_v, idxC_v, idxT_v, pos_v, buf0, buf1, bufW,
          sg0, sg1, sw0, sw1, swW):
    wid = lax.axis_index("s") * NC + lax.axis_index("c")
    base_batch = wid * B_PER_W

    pltpu.sync_copy(idxA_hbm.at[wid], idxA_v)
    pltpu.sync_copy(idxB_hbm.at[wid], idxB_v)
    pltpu.sync_copy(idxC_hbm.at[wid], idxC_v)
    pltpu.sync_copy(idxT_hbm.at[wid], idxT_v)
    pltpu.sync_copy(pos_hbm, pos_v)

    def add_rows(buf, nrows, poff):
        def group_body(g, _):
            rb = g * RG
            for r8 in range(RG):
                r = rb + r8
                for j in range(D_SLICES):
                    sl = pl.ds(j * LANES, LANES)
                    plsc.addupdate(buf.at[r, sl], pos_v[poff + r, sl])
            return 0

        lax.fori_loop(0, nrows // RG, group_body, 0)

    def batch_body(i, _):
        b = base_batch + i

        pltpu.make_async_copy(tab_hbm.at[idxA_v.at[i]], buf0, sg0).wait()
        add_rows(buf0, CP, 0)
        pltpu.async_copy(buf0, out_hbm.at[b, pl.ds(0, CP)], sw0)

        pltpu.make_async_copy(tab_hbm.at[idxB_v.at[i]], buf1, sg1).wait()
        add_rows(buf1, CP, CP)
        pltpu.async_copy(buf1, out_hbm.at[b, pl.ds(CP, CP)], sw1)

        pltpu.make_async_copy(buf0, out_hbm.at[base_batch, pl.ds(0, CP)],
                              sw0).wait()
        pltpu.async_copy(tab_hbm.at[idxC_v.at[i]], buf0, sg0)
        pltpu.make_async_copy(buf1, out_hbm.at[base_batch, pl.ds(CP, CP)],
                              sw1).wait()
        pltpu.async_copy(tab_hbm.at[idxT_v.at[i]], buf1.at[pl.ds(0, CT)], sg1)

        pltpu.make_async_copy(tab_hbm.at[idxC_v.at[i]], buf0, sg0).wait()
        add_rows(buf0, CP, 2 * CP)
        pltpu.async_copy(buf0, out_hbm.at[b, pl.ds(2 * CP, CP)], sw0)

        pltpu.make_async_copy(tab_hbm.at[idxT_v.at[i]],
                              buf1.at[pl.ds(0, CT)], sg1).wait()

        @pl.when(i > 0)
        def _():
            pltpu.make_async_copy(
                bufW, out_hbm.at[base_batch, pl.ds(3 * CP, CW)], swW).wait()

        for k in range(CW):
            for j in range(D_SLICES):
                sl = pl.ds(j * LANES, LANES)
                bufW[k, sl] = buf1[3 + k, sl] + pos_v[3 * CP + k, sl]
        pltpu.async_copy(bufW, out_hbm.at[b, pl.ds(3 * CP, CW)], swW)

        pltpu.make_async_copy(buf0, out_hbm.at[base_batch, pl.ds(2 * CP, CP)],
                              sw0).wait()

        @pl.when(i + 1 < B_PER_W)
        def _():
            pltpu.async_copy(tab_hbm.at[idxA_v.at[i + 1]], buf0, sg0)
            pltpu.async_copy(tab_hbm.at[idxB_v.at[i + 1]], buf1, sg1)

        return 0

    pltpu.async_copy(tab_hbm.at[idxA_v.at[0]], buf0, sg0)
    pltpu.async_copy(tab_hbm.at[idxB_v.at[0]], buf1, sg1)
    lax.fori_loop(0, B_PER_W, batch_body, 0)
    pltpu.make_async_copy(bufW, out_hbm.at[base_batch, pl.ds(3 * CP, CW)],
                          swW).wait()


@jax.jit
def kernel(tokens, embedding_token, embedding_posicao):
    mesh = plsc.VectorSubcoreMesh(core_axis_name="c", subcore_axis_name="s")
    tok = tokens.astype(jnp.int32)
    idxA = tok[:, :CP].reshape(NW, B_PER_W, CP)
    idxB = tok[:, CP:2 * CP].reshape(NW, B_PER_W, CP)
    idxC = tok[:, 2 * CP:3 * CP].reshape(NW, B_PER_W, CP)
    idxT = tok[:, N_TOKEN_ - CT:].reshape(NW, B_PER_W, CT)
    out = pl.kernel(
        _body,
        out_type=jax.ShapeDtypeStruct((BATCH_, N_TOKEN_, N_EMBD_), jnp.float32),
        mesh=mesh,
        scratch_types=[
            pltpu.VMEM((B_PER_W, CP), jnp.int32),
            pltpu.VMEM((B_PER_W, CP), jnp.int32),
            pltpu.VMEM((B_PER_W, CP), jnp.int32),
            pltpu.VMEM((B_PER_W, CT), jnp.int32),
            pltpu.VMEM((N_TOKEN_, N_EMBD_), jnp.float32),
            pltpu.VMEM((CP, N_EMBD_), jnp.float32),
            pltpu.VMEM((CP, N_EMBD_), jnp.float32),
            pltpu.VMEM((CW, N_EMBD_), jnp.float32),
            pltpu.SemaphoreType.DMA,
            pltpu.SemaphoreType.DMA,
            pltpu.SemaphoreType.DMA,
            pltpu.SemaphoreType.DMA,
            pltpu.SemaphoreType.DMA,
        ],
    )(idxA, idxB, idxC, idxT, embedding_token, embedding_posicao)
    return out


# speedup vs baseline: 2.2588x; 1.6723x over previous
import jax
import jax.numpy as jnp
from jax import lax
from jax.experimental import pallas as pl
from jax.experimental.pallas import tpu as pltpu
from jax.experimental.pallas import tpu_sc as plsc

N_VOCAB_ = 49408
N_EMBD_ = 768
N_TOKEN_ = 77
BATCH_ = 1024

NC = 2
NS = 16
LANES = 16
NW = NC * NS

B_PER_W = BATCH_ // NW
NB_CH = 4
TB = 8
NBAND = 10
CH_PER_BAND = B_PER_W // NB_CH
NCH = NBAND * CH_PER_BAND
ROWS = NB_CH * TB
CW = 5
D_SLICES = N_EMBD_ // LANES
NSLOT = (NCH + 2) // 3 * 3


def _body(idx_hbm, tab_hbm, pos_hbm, out_hbm,
          idx_v, pos_v, buf0, buf1, buf2, bufW,
          sg0, sg1, sg2, sw0, sw1, sw2, swW):
    wid = lax.axis_index("s") * NC + lax.axis_index("c")
    base_batch = wid * B_PER_W

    bufs = (buf0, buf1, buf2)
    sgs = (sg0, sg1, sg2)
    sws = (sw0, sw1, sw2)

    pltpu.sync_copy(idx_hbm.at[wid], idx_v)

    def start_gather(q, buf, sem):
        pltpu.async_copy(tab_hbm.at[idx_v.at[q]], buf, sem)

    def step(q, s):
        buf = bufs[s]
        k = q // CH_PER_BAND
        c = lax.rem(q, CH_PER_BAND)
        bb = base_batch + c * NB_CH
        t0 = pl.multiple_of(k * TB, TB)

        @pl.when(q < NCH)
        def _():
            @pl.when(c == 0)
            def _():
                pltpu.sync_copy(pos_hbm.at[pl.ds(t0, TB)], pos_v)

            pltpu.make_async_copy(tab_hbm.at[idx_v.at[q]], buf, sgs[s]).wait()

            @pl.when(k < NBAND - 1)
            def _():
                def add_body(j, _):
                    sl = pl.ds(j * LANES, LANES)
                    for r8 in range(TB):
                        v = pos_v[r8, sl]
                        for i in range(NB_CH):
                            plsc.addupdate(buf.at[i * TB + r8, sl], v)
                    return 0

                lax.fori_loop(0, D_SLICES, add_body, 0)
                for i in range(NB_CH):
                    pltpu.async_copy(buf.at[pl.ds(i * TB, TB)],
                                     out_hbm.at[bb + i, pl.ds(t0, TB)],
                                     sws[s])

            @pl.when(k == NBAND - 1)
            def _():
                @pl.when(q > NCH - CH_PER_BAND)
                def _():
                    for i in range(NB_CH):
                        pltpu.make_async_copy(
                            bufW.at[i],
                            out_hbm.at[base_batch, pl.ds(N_TOKEN_ - CW, CW)],
                            swW).wait()

                def tail_body(j, _):
                    sl = pl.ds(j * LANES, LANES)
                    for r in range(CW):
                        v = pos_v[r, sl]
                        for i in range(NB_CH):
                            bufW[i, r, sl] = buf[i * TB + r, sl] + v
                    return 0

                lax.fori_loop(0, D_SLICES, tail_body, 0)
                for i in range(NB_CH):
                    pltpu.async_copy(bufW.at[i],
                                     out_hbm.at[bb + i,
                                                pl.ds(N_TOKEN_ - CW, CW)],
                                     swW)

        @pl.when(q + 2 < NCH)
        def _():
            s2 = (s + 2) % 3

            @pl.when(jnp.logical_and(q >= 1, q <= NCH - TB))
            def _():
                for i in range(NB_CH):
                    pltpu.make_async_copy(
                        bufs[s2].at[pl.ds(i * TB, TB)],
                        out_hbm.at[base_batch, pl.ds(0, TB)],
                        sws[s2]).wait()

            start_gather(q + 2, bufs[s2], sgs[s2])

    def triple_body(p, _):
        for s in range(3):
            step(p * 3 + s, s)
        return 0

    start_gather(0, buf0, sg0)
    start_gather(1, buf1, sg1)
    lax.fori_loop(0, NSLOT // 3, triple_body, 0)
    for i in range(NB_CH):
        pltpu.make_async_copy(bufW.at[i],
                              out_hbm.at[base_batch, pl.ds(N_TOKEN_ - CW, CW)],
                              swW).wait()


@jax.jit
def kernel(tokens, embedding_token, embedding_posicao):
    mesh = plsc.VectorSubcoreMesh(core_axis_name="c", subcore_axis_name="s")
    tok = tokens.astype(jnp.int32)
    tok_pad = jnp.concatenate([tok, tok[:, N_TOKEN_ - 3:]], axis=1)
    idx = tok_pad.reshape(NW, CH_PER_BAND, NB_CH, NBAND, TB)
    idx = idx.transpose(0, 3, 1, 2, 4).reshape(NW, NCH, ROWS)
    pos_pad = jnp.concatenate(
        [embedding_posicao,
         jnp.zeros((NBAND * TB - N_TOKEN_, N_EMBD_), jnp.float32)], axis=0)
    out = pl.kernel(
        _body,
        out_type=jax.ShapeDtypeStruct((BATCH_, N_TOKEN_, N_EMBD_), jnp.float32),
        mesh=mesh,
        scratch_types=[
            pltpu.VMEM((NCH, ROWS), jnp.int32),
            pltpu.VMEM((TB, N_EMBD_), jnp.float32),
            pltpu.VMEM((ROWS, N_EMBD_), jnp.float32),
            pltpu.VMEM((ROWS, N_EMBD_), jnp.float32),
            pltpu.VMEM((ROWS, N_EMBD_), jnp.float32),
            pltpu.VMEM((NB_CH, CW, N_EMBD_), jnp.float32),
            pltpu.SemaphoreType.DMA,
            pltpu.SemaphoreType.DMA,
            pltpu.SemaphoreType.DMA,
            pltpu.SemaphoreType.DMA,
            pltpu.SemaphoreType.DMA,
            pltpu.SemaphoreType.DMA,
            pltpu.SemaphoreType.DMA,
        ],
    )(idx, embedding_token, pos_pad)
    return out
